# Initial kernel scaffold; baseline (speedup 1.0000x reference)
import jax
import jax.numpy as jnp
from jax.experimental import pallas as pl


def kernel(edge_adjacency_logits, edge_weight_logits, default_distance, reorder, target_paths, from_ix, to_ix):
    raise NotImplementedError("write your pallas kernel here")


# speedup vs baseline: 1.1878x; 1.1878x over previous
import functools

import jax
import jax.numpy as jnp
from jax import lax
from jax.experimental import pallas as pl
from jax.experimental.pallas import tpu as pltpu
from jax.experimental.pallas import tpu_sc as plsc

N_VERT = 100000
N_EDGES = 3200000
E1 = N_EDGES + 1
BATCH = 16384
PATH_LEN = 32

NC, NS = 2, 16
NW = NC * NS
NTOT = BATCH * PATH_LEN
PW = NTOT // NW


def _sc_gather_body(tp_hbm, reorder_hbm, w_hbm, a_hbm, wout_hbm, aout_hbm,
                    idx_v, pidx_v, w_v, a_v, sem):
    wid = lax.axis_index("s") * NC + lax.axis_index("c")
    base = wid * PW
    pltpu.sync_copy(tp_hbm.at[pl.ds(base, PW)], idx_v)
    pltpu.async_copy(reorder_hbm.at[idx_v], pidx_v, sem).wait()
    pltpu.async_copy(w_hbm.at[pidx_v], w_v, sem).wait()
    pltpu.async_copy(a_hbm.at[pidx_v], a_v, sem).wait()
    pltpu.sync_copy(w_v, wout_hbm.at[pl.ds(base, PW)])
    pltpu.sync_copy(a_v, aout_hbm.at[pl.ds(base, PW)])


_sc_gather = functools.partial(
    pl.kernel,
    out_type=(jax.ShapeDtypeStruct((NTOT,), jnp.float32),
              jax.ShapeDtypeStruct((NTOT,), jnp.float32)),
    mesh=plsc.VectorSubcoreMesh(core_axis_name="c", subcore_axis_name="s"),
    scratch_types=[
        pltpu.VMEM((PW,), jnp.int32),
        pltpu.VMEM((PW,), jnp.int32),
        pltpu.VMEM((PW,), jnp.float32),
        pltpu.VMEM((PW,), jnp.float32),
        pltpu.SemaphoreType.DMA,
    ],
)(_sc_gather_body)


def _softplus(x):
    return jnp.maximum(x, 0.0) + jnp.log1p(jnp.exp(-jnp.abs(x)))


def _tc_epilogue_body(w_ref, a_ref, tp_ref, fi_ref, ti_ref, dflt_ref,
                      dist_ref, logp_ref):
    w = w_ref[...]
    a = a_ref[...]
    tp = tp_ref[...]
    valid = (tp != 0)
    zero = jnp.zeros_like(w)
    sp_w = jnp.where(valid, _softplus(w), zero)
    sp_a = jnp.where(valid, _softplus(-a), zero)
    dist = jnp.sum(sp_w, axis=0)
    logp = -jnp.sum(sp_a, axis=0)
    not_found = (tp[0, :] == 0) & (fi_ref[...] != ti_ref[...])
    dist_ref[...] = jnp.where(not_found, dflt_ref[0, 0], dist)
    logp_ref[...] = logp


def _tc_epilogue(w_t, a_t, tp_t, fi, ti, dflt):
    bs = 2048
    grid = (BATCH // bs,)
    spec2d = pl.BlockSpec((PATH_LEN, bs), lambda i: (0, i))
    spec1d = pl.BlockSpec((bs,), lambda i: (i,))
    spec0 = pl.BlockSpec((1, 1), lambda i: (0, 0))
    return pl.pallas_call(
        _tc_epilogue_body,
        grid=grid,
        in_specs=[spec2d, spec2d, spec2d, spec1d, spec1d, spec0],
        out_specs=[spec1d, spec1d],
        out_shape=(jax.ShapeDtypeStruct((BATCH,), jnp.float32),
                   jax.ShapeDtypeStruct((BATCH,), jnp.float32)),
    )(w_t, a_t, tp_t, fi, ti, dflt)


def kernel(edge_adjacency_logits, edge_weight_logits, default_distance,
           reorder, target_paths, from_ix, to_ix):
    reorder_i = reorder.astype(jnp.int32)
    tp_t = target_paths.astype(jnp.int32).T
    tp_flat = tp_t.reshape(NTOT)
    w_tab = edge_weight_logits.reshape(E1)
    a_tab = edge_adjacency_logits.reshape(E1)
    w_g, a_g = _sc_gather(tp_flat, reorder_i, w_tab, a_tab)
    w_gt = w_g.reshape(PATH_LEN, BATCH)
    a_gt = a_g.reshape(PATH_LEN, BATCH)
    dist, logp = _tc_epilogue(w_gt, a_gt, tp_t,
                              from_ix.astype(jnp.int32),
                              to_ix.astype(jnp.int32),
                              default_distance)
    return dist, logp
